# Initial kernel scaffold; baseline (speedup 1.0000x reference)
import jax
import jax.numpy as jnp
from jax.experimental import pallas as pl


def kernel(x, W):
    raise NotImplementedError("write your pallas kernel here")


# speedup vs baseline: 3.7062x; 3.7062x over previous
import functools

import jax
import jax.numpy as jnp
from jax.experimental import pallas as pl
from jax.experimental.pallas import tpu as pltpu

_TB = 512


def _router_block(x_ref, wt_ref, out_ref, idx_ref):
    l = jax.lax.dot_general(
        x_ref[...], wt_ref[...], (((1,), (0,)), ((), ())),
        preferred_element_type=jnp.float32,
    )
    iota = jax.lax.broadcasted_iota(jnp.int32, l.shape, 1)
    m1 = jnp.max(l, axis=1, keepdims=True)
    idx1 = jnp.min(jnp.where(l == m1, iota, 64), axis=1, keepdims=True)
    hit1 = iota == idx1
    l2 = jnp.where(hit1, -jnp.inf, l)
    m2 = jnp.max(l2, axis=1, keepdims=True)
    idx2 = jnp.min(jnp.where(l2 == m2, iota, 64), axis=1, keepdims=True)
    e = jnp.exp(l - m1)
    s = e / jnp.sum(e, axis=1, keepdims=True)
    out_ref[...] = jnp.where(hit1 | (iota == idx2), s, 0.0)
    idx_ref[...] = jnp.concatenate([idx1, idx2], axis=1).astype(jnp.int32)


@jax.jit
def kernel(x, W):
    B, T, C = x.shape
    E = W.shape[0]
    N = B * T
    xf = x.reshape(N, C)
    wt = W.T
    grid = (N // _TB,)
    out, idx = pl.pallas_call(
        _router_block,
        grid=grid,
        in_specs=[
            pl.BlockSpec((_TB, C), lambda i: (i, 0)),
            pl.BlockSpec((C, E), lambda i: (0, 0)),
        ],
        out_specs=[
            pl.BlockSpec((_TB, E), lambda i: (i, 0)),
            pl.BlockSpec((_TB, 2), lambda i: (i, 0)),
        ],
        out_shape=[
            jax.ShapeDtypeStruct((N, E), jnp.float32),
            jax.ShapeDtypeStruct((N, 2), jnp.int32),
        ],
        compiler_params=pltpu.CompilerParams(
            dimension_semantics=("arbitrary",),
        ),
    )(xf, wt)
    return out.reshape(B, T, E), idx.reshape(B, T, 2)


# speedup vs baseline: 3.9063x; 1.0540x over previous
import functools

import jax
import jax.numpy as jnp
from jax.experimental import pallas as pl
from jax.experimental.pallas import tpu as pltpu

_TB = 512


def _router_block(x_ref, wt_ref, out_ref, idx_ref):
    l = jax.lax.dot_general(
        x_ref[...], wt_ref[...], (((1,), (0,)), ((), ())),
        preferred_element_type=jnp.float32,
    )
    iota = jax.lax.broadcasted_iota(jnp.int32, l.shape, 1).astype(jnp.float32)
    m1 = jnp.max(l, axis=1, keepdims=True)
    idx1 = jnp.min(jnp.where(l == m1, iota, 64.0), axis=1, keepdims=True)
    hit1 = iota == idx1
    l2 = jnp.where(hit1, -jnp.inf, l)
    m2 = jnp.max(l2, axis=1, keepdims=True)
    idx2 = jnp.min(jnp.where(l2 == m2, iota, 64.0), axis=1, keepdims=True)
    e = jnp.exp(l - m1)
    sinv = 1.0 / jnp.sum(e, axis=1, keepdims=True)
    out_ref[...] = jnp.where(hit1 | (iota == idx2), e * sinv, 0.0)
    idx_ref[...] = jnp.concatenate([idx1, idx2], axis=1).astype(jnp.int32)


@jax.jit
def kernel(x, W):
    B, T, C = x.shape
    E = W.shape[0]
    N = B * T
    xf = x.reshape(N, C)
    wt = W.T
    grid = (N // _TB,)
    out, idx = pl.pallas_call(
        _router_block,
        grid=grid,
        in_specs=[
            pl.BlockSpec((_TB, C), lambda i: (i, 0)),
            pl.BlockSpec((C, E), lambda i: (0, 0)),
        ],
        out_specs=[
            pl.BlockSpec((_TB, E), lambda i: (i, 0)),
            pl.BlockSpec((_TB, 2), lambda i: (i, 0)),
        ],
        out_shape=[
            jax.ShapeDtypeStruct((N, E), jnp.float32),
            jax.ShapeDtypeStruct((N, 2), jnp.int32),
        ],
        compiler_params=pltpu.CompilerParams(
            dimension_semantics=("arbitrary",),
        ),
    )(xf, wt)
    return out.reshape(B, T, E), idx.reshape(B, T, 2)


# speedup vs baseline: 4.1838x; 1.0710x over previous
import functools

import jax
import jax.numpy as jnp
from jax.experimental import pallas as pl
from jax.experimental.pallas import tpu as pltpu

_TB = 512


def _router_block(x_ref, w_ref, out_ref, idx_ref):
    l = jax.lax.dot_general(
        x_ref[0], w_ref[...], (((1,), (1,)), ((), ())),
        preferred_element_type=jnp.float32,
    )
    iota = jax.lax.broadcasted_iota(jnp.int32, l.shape, 1).astype(jnp.float32)
    m1 = jnp.max(l, axis=1, keepdims=True)
    idx1 = jnp.min(jnp.where(l == m1, iota, 64.0), axis=1, keepdims=True)
    hit1 = iota == idx1
    l2 = jnp.where(hit1, -jnp.inf, l)
    m2 = jnp.max(l2, axis=1, keepdims=True)
    idx2 = jnp.min(jnp.where(l2 == m2, iota, 64.0), axis=1, keepdims=True)
    e = jnp.exp(l - m1)
    sinv = 1.0 / jnp.sum(e, axis=1, keepdims=True)
    out_ref[0] = jnp.where(hit1 | (iota == idx2), e * sinv, 0.0)
    idx_ref[0] = jnp.concatenate([idx1, idx2], axis=1).astype(jnp.int32)


@jax.jit
def kernel(x, W):
    B, T, C = x.shape
    E = W.shape[0]
    grid = (B, T // _TB)
    out, idx = pl.pallas_call(
        _router_block,
        grid=grid,
        in_specs=[
            pl.BlockSpec((1, _TB, C), lambda b, i: (b, i, 0)),
            pl.BlockSpec((E, C), lambda b, i: (0, 0)),
        ],
        out_specs=[
            pl.BlockSpec((1, _TB, E), lambda b, i: (b, i, 0)),
            pl.BlockSpec((1, _TB, 2), lambda b, i: (b, i, 0)),
        ],
        out_shape=[
            jax.ShapeDtypeStruct((B, T, E), jnp.float32),
            jax.ShapeDtypeStruct((B, T, 2), jnp.int32),
        ],
        compiler_params=pltpu.CompilerParams(
            dimension_semantics=("arbitrary", "arbitrary"),
        ),
    )(x, W)
    return out, idx


# speedup vs baseline: 5.2378x; 1.2519x over previous
import functools

import jax
import jax.numpy as jnp
from jax.experimental import pallas as pl
from jax.experimental.pallas import tpu as pltpu

_TB = 1024


def _router_block(x_ref, w_ref, out_ref, idx_ref):
    l = jax.lax.dot_general(
        x_ref[0], w_ref[...], (((1,), (1,)), ((), ())),
        preferred_element_type=jnp.float32,
    )
    iota = jax.lax.broadcasted_iota(jnp.int32, l.shape, 1).astype(jnp.float32)
    m1 = jnp.max(l, axis=1, keepdims=True)
    idx1 = jnp.min(jnp.where(l == m1, iota, 64.0), axis=1, keepdims=True)
    hit1 = iota == idx1
    l2 = jnp.where(hit1, -jnp.inf, l)
    m2 = jnp.max(l2, axis=1, keepdims=True)
    idx2 = jnp.min(jnp.where(l2 == m2, iota, 64.0), axis=1, keepdims=True)
    e = jnp.exp(l - m1)
    sinv = 1.0 / jnp.sum(e, axis=1, keepdims=True)
    out_ref[0] = jnp.where(hit1 | (iota == idx2), e * sinv, 0.0)
    idx_ref[0] = jnp.concatenate([idx1, idx2], axis=1).astype(jnp.int32)


@jax.jit
def kernel(x, W):
    B, T, C = x.shape
    E = W.shape[0]
    grid = (B, T // _TB)
    out, idx = pl.pallas_call(
        _router_block,
        grid=grid,
        in_specs=[
            pl.BlockSpec((1, _TB, C), lambda b, i: (b, i, 0)),
            pl.BlockSpec((E, C), lambda b, i: (0, 0)),
        ],
        out_specs=[
            pl.BlockSpec((1, _TB, E), lambda b, i: (b, i, 0)),
            pl.BlockSpec((1, _TB, 2), lambda b, i: (b, i, 0)),
        ],
        out_shape=[
            jax.ShapeDtypeStruct((B, T, E), jnp.float32),
            jax.ShapeDtypeStruct((B, T, 2), jnp.int32),
        ],
        compiler_params=pltpu.CompilerParams(
            dimension_semantics=("arbitrary", "arbitrary"),
        ),
    )(x, W)
    return out, idx


# speedup vs baseline: 5.8035x; 1.1080x over previous
import functools

import jax
import jax.numpy as jnp
from jax.experimental import pallas as pl
from jax.experimental.pallas import tpu as pltpu

_TB = 2048


def _router_block(x_ref, w_ref, out_ref, idx_ref):
    l = jax.lax.dot_general(
        x_ref[0], w_ref[...], (((1,), (1,)), ((), ())),
        preferred_element_type=jnp.float32,
    )
    iota = jax.lax.broadcasted_iota(jnp.int32, l.shape, 1).astype(jnp.float32)
    m1 = jnp.max(l, axis=1, keepdims=True)
    idx1 = jnp.min(jnp.where(l == m1, iota, 64.0), axis=1, keepdims=True)
    hit1 = iota == idx1
    l2 = jnp.where(hit1, -jnp.inf, l)
    m2 = jnp.max(l2, axis=1, keepdims=True)
    idx2 = jnp.min(jnp.where(l2 == m2, iota, 64.0), axis=1, keepdims=True)
    e = jnp.exp(l - m1)
    sinv = 1.0 / jnp.sum(e, axis=1, keepdims=True)
    out_ref[0] = jnp.where(hit1 | (iota == idx2), e * sinv, 0.0)
    idx_ref[0] = jnp.concatenate([idx1, idx2], axis=1).astype(jnp.int32)


@jax.jit
def kernel(x, W):
    B, T, C = x.shape
    E = W.shape[0]
    grid = (B, T // _TB)
    out, idx = pl.pallas_call(
        _router_block,
        grid=grid,
        in_specs=[
            pl.BlockSpec((1, _TB, C), lambda b, i: (b, i, 0)),
            pl.BlockSpec((E, C), lambda b, i: (0, 0)),
        ],
        out_specs=[
            pl.BlockSpec((1, _TB, E), lambda b, i: (b, i, 0)),
            pl.BlockSpec((1, _TB, 2), lambda b, i: (b, i, 0)),
        ],
        out_shape=[
            jax.ShapeDtypeStruct((B, T, E), jnp.float32),
            jax.ShapeDtypeStruct((B, T, 2), jnp.int32),
        ],
        compiler_params=pltpu.CompilerParams(
            dimension_semantics=("arbitrary", "arbitrary"),
        ),
    )(x, W)
    return out, idx


# speedup vs baseline: 6.0448x; 1.0416x over previous
import functools

import jax
import jax.numpy as jnp
from jax.experimental import pallas as pl
from jax.experimental.pallas import tpu as pltpu

_TB = 4096


def _router_block(x_ref, w_ref, out_ref, idx_ref):
    l = jax.lax.dot_general(
        x_ref[0], w_ref[...], (((1,), (1,)), ((), ())),
        preferred_element_type=jnp.float32,
    )
    iota = jax.lax.broadcasted_iota(jnp.int32, l.shape, 1).astype(jnp.float32)
    m1 = jnp.max(l, axis=1, keepdims=True)
    idx1 = jnp.min(jnp.where(l == m1, iota, 64.0), axis=1, keepdims=True)
    hit1 = iota == idx1
    l2 = jnp.where(hit1, -jnp.inf, l)
    m2 = jnp.max(l2, axis=1, keepdims=True)
    idx2 = jnp.min(jnp.where(l2 == m2, iota, 64.0), axis=1, keepdims=True)
    e = jnp.exp(l - m1)
    sinv = 1.0 / jnp.sum(e, axis=1, keepdims=True)
    out_ref[0] = jnp.where(hit1 | (iota == idx2), e * sinv, 0.0)
    idx_ref[0] = jnp.concatenate([idx1, idx2], axis=1).astype(jnp.int32)


@jax.jit
def kernel(x, W):
    B, T, C = x.shape
    E = W.shape[0]
    grid = (B, T // _TB)
    out, idx = pl.pallas_call(
        _router_block,
        grid=grid,
        in_specs=[
            pl.BlockSpec((1, _TB, C), lambda b, i: (b, i, 0)),
            pl.BlockSpec((E, C), lambda b, i: (0, 0)),
        ],
        out_specs=[
            pl.BlockSpec((1, _TB, E), lambda b, i: (b, i, 0)),
            pl.BlockSpec((1, _TB, 2), lambda b, i: (b, i, 0)),
        ],
        out_shape=[
            jax.ShapeDtypeStruct((B, T, E), jnp.float32),
            jax.ShapeDtypeStruct((B, T, 2), jnp.int32),
        ],
        compiler_params=pltpu.CompilerParams(
            dimension_semantics=("arbitrary", "arbitrary"),
        ),
    )(x, W)
    return out, idx
